# baseline (speedup 1.0000x reference)
import functools

import jax
import jax.numpy as jnp
import numpy as np
from jax import lax
from jax.experimental import pallas as pl
from jax.experimental.pallas import tpu as pltpu
from jax.experimental.pallas import tpu_sc as plsc

_BIG = 1e10
_B = 16
_N = 512
_D = 64
_W = 528
_LP = 1056
_CH = 48
_NCH = _LP // _CH

_SARR = np.arange(_W)
_PV = 33 * (_SARR % 16) + _SARR // 16
_IVEC = np.clip(_PV - 1, 0, _N - 1).astype(np.int32)
_ANS_LANE = int(np.where(_PV == _N)[0][0])

_LOG1P = (0.008592109931055492, -0.06303373373867692, 0.2067238479723716,
          -0.4512964175334485, 0.9917296877716534, 0.0003529662470068695)


def _softmin3(a, b, c):
    m1 = jnp.minimum(a, b)
    mx1 = jnp.maximum(a, b)
    mn = jnp.minimum(m1, c)
    mx2 = jnp.maximum(m1, c)
    u = jnp.exp(mn - mx2) + jnp.exp(mn - mx1)
    c5, c4, c3, c2, c1, c0 = _LOG1P
    u2 = u * u
    pol = (c5 * u + c4) * u2 + ((c3 * u + c2) * u2 + (c1 * u + c0))
    return mn - pol


def _skew_body(xg_ref, y_ref, mm_ref):
    xb = xg_ref[0]
    yb = y_ref[0]
    xn = jnp.sum(xb * xb, axis=1)
    yn = jnp.sum(yb * yb, axis=1)
    g = lax.dot_general(yb, xb, (((1,), (1,)), ((), ())),
                        preferred_element_type=jnp.float32)
    p = yn[:, None] + xn[None, :] - 2.0 * g
    p = jnp.clip(p, 0.0, None)
    s2 = lax.broadcasted_iota(jnp.int32, (_N, _W), 1)
    pvv = 33 * (s2 & 15) + (s2 >> 4)
    p = jnp.where((pvv < 1) | (pvv > _N), _BIG, p)
    p = jnp.concatenate(
        [p, jnp.full((_LP - _N, _W), _BIG, jnp.float32)], axis=0)
    s3 = lax.broadcasted_iota(jnp.int32, (_LP, _W), 1)
    rv = 33 * (s3 & 15) + (s3 >> 4) - 1
    for k in range(9):
        sh = 1 << k
        rolled = jnp.concatenate([p[_LP - sh:], p[:_LP - sh]], axis=0)
        p = jnp.where((rv & sh) != 0, rolled, p)
    mm_ref[0] = p


def _skewed_distances(x, y):
    xg = jnp.take(x, jnp.asarray(_IVEC), axis=1)
    return pl.pallas_call(
        _skew_body,
        grid=(_B,),
        in_specs=[
            pl.BlockSpec((1, _W, _D), lambda b: (b, 0, 0)),
            pl.BlockSpec((1, _N, _D), lambda b: (b, 0, 0)),
        ],
        out_specs=pl.BlockSpec((1, _LP, _W), lambda b: (b, 0, 0)),
        out_shape=jax.ShapeDtypeStruct((_B, _LP, _W), jnp.float32),
    )(xg, y)


def _sdtw_sc(mm):
    mesh = plsc.VectorSubcoreMesh(core_axis_name="c", subcore_axis_name="s")

    @functools.partial(
        pl.kernel,
        out_type=jax.ShapeDtypeStruct((_B, 16), jnp.float32),
        mesh=mesh,
        scratch_types=[
            pltpu.VMEM((_CH, _W), jnp.float32),
            pltpu.VMEM((_CH, _W), jnp.float32),
            pltpu.VMEM((16 + _W,), jnp.float32),
            pltpu.VMEM((16 + _W,), jnp.float32),
            pltpu.VMEM((16 + _W,), jnp.float32),
            pltpu.VMEM((16,), jnp.float32),
            pltpu.SemaphoreType.DMA,
            pltpu.SemaphoreType.DMA,
        ],
    )
    def run(mm_hbm, out_hbm, buf0, buf1, w0, w1, w2, ovec, sem0, sem1):
        wid = lax.axis_index("c") * 16 + lax.axis_index("s")

        @pl.when(wid < _B)
        def _():
            b = wid
            it = lax.iota(jnp.int32, 16)
            m0 = it == 0
            bigv = jnp.full((16,), _BIG, jnp.float32)

            def copy_chunk(c, buf, sem):
                return pltpu.make_async_copy(
                    mm_hbm.at[b, pl.ds(c * _CH, _CH)], buf, sem)

            def set_halo(w):
                w[pl.ds(0, 16)] = jnp.where(
                    m0, bigv, w[pl.ds(16 + _N - 1, 16)])

            def do_step(wn, wa, wb, buf, lloc):
                @plsc.parallel_loop(16, 16 + _W, step=16, unroll=16)
                def _(soff):
                    diag = wb[pl.ds(soff - 16, 16)]
                    up = wa[pl.ds(soff - 16, 16)]
                    left = wa[pl.ds(soff, 16)]
                    mmv = buf[lloc, pl.ds(soff - 16, 16)]
                    wn[pl.ds(soff, 16)] = _softmin3(diag, up, left) + mmv

                set_halo(wn)

            def do_triples(base, ntrip, buf, coff):
                def triple(t, _):
                    lb = base + 3 * t
                    do_step(w0, w2, w1, buf, lb - coff)
                    do_step(w1, w0, w2, buf, lb + 1 - coff)
                    do_step(w2, w1, w0, buf, lb + 2 - coff)
                    return 0
                lax.fori_loop(0, ntrip, triple, 0)

            cp0 = copy_chunk(0, buf0, sem0)
            cp0.start()
            cp0.wait()
            copy_chunk(1, buf1, sem1).start()

            mm00 = jnp.full((16,), buf0[0, pl.ds(16, 16)][0], jnp.float32)
            for j in range(_W // 16):
                off = j * 16
                w0[pl.ds(16 + off, 16)] = buf0[0, pl.ds(off, 16)]
                w1[pl.ds(16 + off, 16)] = buf0[1, pl.ds(off, 16)] + mm00
            set_halo(w0)
            set_halo(w1)
            do_step(w2, w1, w0, buf0, 2)
            do_triples(3, (_CH - 3) // 3, buf0, 0)
            copy_chunk(2, buf0, sem0).start()
            cp1w = copy_chunk(1, buf1, sem1)
            cp1w.wait()

            def pair(t, _):
                c = 2 * t + 1
                do_triples(c * _CH, jnp.where(c == _NCH - 1, 5, _CH // 3),
                           buf1, c * _CH)

                @pl.when(c + 2 < _NCH)
                def _():
                    copy_chunk(c + 2, buf1, sem1).start()

                @pl.when(c + 1 < _NCH)
                def _():
                    copy_chunk(c + 1, buf0, sem0).wait()
                    do_triples((c + 1) * _CH, _CH // 3, buf0, (c + 1) * _CH)

                @pl.when(c + 3 < _NCH)
                def _():
                    copy_chunk(c + 3, buf0, sem0).start()

                @pl.when(c + 2 < _NCH)
                def _():
                    copy_chunk(c + 2, buf1, sem1).wait()
                return 0

            lax.fori_loop(0, _NCH // 2, pair, 0)

            ovec[...] = w2[pl.ds(16 + _ANS_LANE - 15, 16)]
            pltpu.sync_copy(ovec, out_hbm.at[b])

    return run(mm)


def kernel(x, y):
    mm = _skewed_distances(x, y)
    out16 = _sdtw_sc(mm)
    return out16[:, 15]


# speedup vs baseline: 1.6034x; 1.6034x over previous
import functools

import jax
import jax.numpy as jnp
import numpy as np
from jax import lax
from jax.experimental import pallas as pl
from jax.experimental.pallas import tpu as pltpu
from jax.experimental.pallas import tpu_sc as plsc

_BIG = 1e10
_B = 16
_N = 512
_D = 64
_W = 528
_LP = 1056
_CH = 48
_NCH = _LP // _CH

_SARR = np.arange(_W)
_PV = 33 * (_SARR % 16) + _SARR // 16
_IVEC = np.clip(_PV - 1, 0, _N - 1).astype(np.int32)
_ANS_LANE = int(np.where(_PV == _N)[0][0])

_LOG1P = (0.008592109931055492, -0.06303373373867692, 0.2067238479723716,
          -0.4512964175334485, 0.9917296877716534, 0.0003529662470068695)


def _softmin3_add(a, b, c, add):
    m1 = jnp.minimum(a, b)
    mx1 = jnp.maximum(a, b)
    mn = jnp.minimum(m1, c)
    mx2 = jnp.maximum(m1, c)
    u = jnp.exp(mn - mx2) + jnp.exp(mn - mx1)
    c5, c4, c3, c2, c1, c0 = _LOG1P
    u2 = u * u
    pol = (c5 * u + c4) * u2 + ((c3 * u + c2) * u2 + (c1 * u + c0))
    return (mn + add) - pol


def _skew_body(xg_ref, y_ref, mm_ref):
    xb = xg_ref[0]
    yb = y_ref[0]
    xn = jnp.sum(xb * xb, axis=1)
    yn = jnp.sum(yb * yb, axis=1)
    g = lax.dot_general(yb, xb, (((1,), (1,)), ((), ())),
                        preferred_element_type=jnp.float32)
    p = yn[:, None] + xn[None, :] - 2.0 * g
    p = jnp.clip(p, 0.0, None)
    s2 = lax.broadcasted_iota(jnp.int32, (_N, _W), 1)
    pvv = 33 * (s2 & 15) + (s2 >> 4)
    p = jnp.where((pvv < 1) | (pvv > _N), _BIG, p)
    p = jnp.concatenate(
        [p, jnp.full((_LP - _N, _W), _BIG, jnp.float32)], axis=0)
    s3 = lax.broadcasted_iota(jnp.int32, (_LP, _W), 1)
    rv = 33 * (s3 & 15) + (s3 >> 4) - 1
    for k in range(9):
        sh = 1 << k
        rolled = jnp.concatenate([p[_LP - sh:], p[:_LP - sh]], axis=0)
        p = jnp.where((rv & sh) != 0, rolled, p)
    mm_ref[0] = p


def _skewed_distances(x, y):
    xg = jnp.take(x, jnp.asarray(_IVEC), axis=1)
    return pl.pallas_call(
        _skew_body,
        grid=(_B,),
        in_specs=[
            pl.BlockSpec((1, _W, _D), lambda b: (b, 0, 0)),
            pl.BlockSpec((1, _N, _D), lambda b: (b, 0, 0)),
        ],
        out_specs=pl.BlockSpec((1, _LP, _W), lambda b: (b, 0, 0)),
        out_shape=jax.ShapeDtypeStruct((_B, _LP, _W), jnp.float32),
    )(xg, y)


def _sdtw_sc(mm):
    mesh = plsc.VectorSubcoreMesh(core_axis_name="c", subcore_axis_name="s")

    @functools.partial(
        pl.kernel,
        out_type=jax.ShapeDtypeStruct((_B, 16), jnp.float32),
        mesh=mesh,
        scratch_types=[
            pltpu.VMEM((_CH, _W), jnp.float32),
            pltpu.VMEM((_CH, _W), jnp.float32),
            pltpu.VMEM((16 + _W,), jnp.float32),
            pltpu.VMEM((16 + _W,), jnp.float32),
            pltpu.VMEM((16 + _W,), jnp.float32),
            pltpu.VMEM((16,), jnp.float32),
            pltpu.SemaphoreType.DMA,
            pltpu.SemaphoreType.DMA,
        ],
    )
    def run(mm_hbm, out_hbm, buf0, buf1, w0, w1, w2, ovec, sem0, sem1):
        wid = lax.axis_index("c") * 16 + lax.axis_index("s")

        @pl.when(wid < _B)
        def _():
            b = wid
            it = lax.iota(jnp.int32, 16)
            m0 = it == 0
            bigv = jnp.full((16,), _BIG, jnp.float32)

            def copy_chunk(c, buf, sem):
                return pltpu.make_async_copy(
                    mm_hbm.at[b, pl.ds(c * _CH, _CH)], buf, sem)

            def set_halo(w):
                w[pl.ds(0, 16)] = jnp.where(
                    m0, bigv, w[pl.ds(16 + _N - 1, 16)])

            def do_step(wn, wa, wb, buf, lloc):
                @plsc.parallel_loop(16, 16 + _W, step=16, unroll=11)
                def _(soff):
                    diag = wb[pl.ds(soff - 16, 16)]
                    up = wa[pl.ds(soff - 16, 16)]
                    left = wa[pl.ds(soff, 16)]
                    mmv = buf[lloc, pl.ds(soff - 16, 16)]
                    wn[pl.ds(soff, 16)] = _softmin3_add(diag, up, left, mmv)

                set_halo(wn)

            def do_triples(base, ntrip, buf, coff):
                def triple(t, _):
                    lb = base + 3 * t
                    do_step(w0, w2, w1, buf, lb - coff)
                    do_step(w1, w0, w2, buf, lb + 1 - coff)
                    do_step(w2, w1, w0, buf, lb + 2 - coff)
                    return 0
                lax.fori_loop(0, ntrip, triple, 0)

            cp0 = copy_chunk(0, buf0, sem0)
            cp0.start()
            cp0.wait()
            copy_chunk(1, buf1, sem1).start()

            mm00 = jnp.full((16,), buf0[0, pl.ds(16, 16)][0], jnp.float32)
            for j in range(_W // 16):
                off = j * 16
                w0[pl.ds(16 + off, 16)] = buf0[0, pl.ds(off, 16)]
                w1[pl.ds(16 + off, 16)] = buf0[1, pl.ds(off, 16)] + mm00
            set_halo(w0)
            set_halo(w1)
            do_step(w2, w1, w0, buf0, 2)
            do_triples(3, (_CH - 3) // 3, buf0, 0)
            copy_chunk(2, buf0, sem0).start()
            cp1w = copy_chunk(1, buf1, sem1)
            cp1w.wait()

            def pair(t, _):
                c = 2 * t + 1
                do_triples(c * _CH, jnp.where(c == _NCH - 1, 5, _CH // 3),
                           buf1, c * _CH)

                @pl.when(c + 2 < _NCH)
                def _():
                    copy_chunk(c + 2, buf1, sem1).start()

                @pl.when(c + 1 < _NCH)
                def _():
                    copy_chunk(c + 1, buf0, sem0).wait()
                    do_triples((c + 1) * _CH, _CH // 3, buf0, (c + 1) * _CH)

                @pl.when(c + 3 < _NCH)
                def _():
                    copy_chunk(c + 3, buf0, sem0).start()

                @pl.when(c + 2 < _NCH)
                def _():
                    copy_chunk(c + 2, buf1, sem1).wait()
                return 0

            lax.fori_loop(0, _NCH // 2, pair, 0)

            ovec[...] = w2[pl.ds(16 + _ANS_LANE - 15, 16)]
            pltpu.sync_copy(ovec, out_hbm.at[b])

    return run(mm)


def kernel(x, y):
    mm = _skewed_distances(x, y)
    out16 = _sdtw_sc(mm)
    return out16[:, 15]
